# baseline (speedup 1.0000x reference)
import functools

import jax
import jax.numpy as jnp
from jax import lax
from jax.experimental import pallas as pl
from jax.experimental.pallas import tpu as pltpu
from jax.experimental.pallas import tpu_sc as plsc

N = 10000
E = 320000
D = 128
DE = 16

NC = 2
NS = 16
NW = NC * NS
EW = E // NW
GCH = 125
GNB = 2
GNCH = EW // GCH
CHUNK = 80
NCHUNK = EW // CHUNK
NBUF = 3
N_SH = 10080
ZCH = 80
NZ = N_SH // ZCH
OUT_CHUNK = 80
NOUT = N // OUT_CHUNK

_vector_mesh = plsc.VectorSubcoreMesh(core_axis_name="c", subcore_axis_name="s")


def _zero_rows(ref):
    zv = jnp.zeros((16,), jnp.float32)

    @pl.loop(0, ref.shape[0])
    def _(i):
        @pl.loop(0, D, step=16)
        def _(j):
            ref[i, pl.ds(j, 16)] = zv


def _zero_accum(zb, sh_ref, s):
    @pl.loop(0, pl.cdiv(NZ, NS))
    def _(k):
        ci = k * NS + s

        @pl.when(ci < NZ)
        def _():
            pltpu.sync_copy(zb, sh_ref.at[pl.ds(ci * ZCH, ZCH)])


def _copy_out(sh_ref, out_ref, c, s):
    @pl.loop(0, pl.cdiv(NOUT, NS))
    def _(k):
        ci = k * NS + s

        @pl.when(ci < NOUT)
        def _():
            row = ci * OUT_CHUNK
            pltpu.sync_copy(sh_ref.at[pl.ds(row, OUT_CHUNK)],
                            out_ref.at[c, pl.ds(row, OUT_CHUNK)])


def _edge_pipeline(nchunk, nbuf, idx_issue, idx_wait, load_issue, load_wait,
                   rows_v, dsts, g_sh, ssems):
    def s_issue(kk, b):
        pltpu.async_copy(rows_v.at[b], g_sh.at[dsts[b]], ssems[b], add=True)

    def s_wait(kk, b):
        pltpu.make_async_copy(rows_v.at[b], g_sh.at[dsts[b]], ssems[b]).wait()

    for b in range(nbuf):
        idx_issue(b, b)

    @pl.loop(0, pl.cdiv(nchunk, nbuf))
    def _(g):
        for b in range(nbuf):
            kk = g * nbuf + b

            @pl.when(kk < nchunk)
            def _():
                idx_wait(kk, b)
                load_issue(kk, b)

        for b in range(nbuf):
            kk = g * nbuf + b

            @pl.when(kk < nchunk)
            def _():
                load_wait(kk, b)
                s_issue(kk, b)

        for b in range(nbuf):
            kk = g * nbuf + b
            nxt = kk + nbuf

            @pl.when(kk < nchunk)
            def _():
                s_wait(kk, b)

                @pl.when(nxt < nchunk)
                def _():
                    idx_issue(nxt, b)


@functools.partial(
    pl.kernel,
    out_type=jax.ShapeDtypeStruct((NC, N, D), jnp.float32),
    mesh=_vector_mesh,
    scratch_types=[
        pltpu.VMEM((GCH,), jnp.int32),
        pltpu.VMEM((GCH,), jnp.int32),
        pltpu.VMEM((GCH,), jnp.int32),
        pltpu.VMEM((GCH,), jnp.int32),
        pltpu.VMEM((GNB, GCH, D), jnp.float32),
        pltpu.VMEM_SHARED((N_SH, D), jnp.float32),
        pltpu.SemaphoreType.DMA,
        pltpu.SemaphoreType.DMA,
        pltpu.SemaphoreType.DMA,
        pltpu.SemaphoreType.DMA,
        pltpu.SemaphoreType.DMA,
        pltpu.SemaphoreType.DMA,
    ],
)
def _sc_gather_segsum(x_hbm, ei_hbm, g_out,
                      s0, s1, d0, d1, rows_v, g_sh,
                      is0, is1, gs0, gs1, ss0, ss1):
    c = lax.axis_index("c")
    s = lax.axis_index("s")
    wid = c * NS + s
    srcs = (s0, s1)
    dsts = (d0, d1)
    isems = (is0, is1)
    gsems = (gs0, gs1)
    ssems = (ss0, ss1)

    _zero_rows(rows_v.at[0])
    _zero_accum(rows_v.at[0].at[pl.ds(0, ZCH)], g_sh, s)
    plsc.subcore_barrier()

    def i_issue(kk, b):
        pltpu.async_copy(ei_hbm.at[0, wid, kk], srcs[b], isems[b])
        pltpu.async_copy(ei_hbm.at[1, wid, kk], dsts[b], isems[b])

    def i_wait(kk, b):
        pltpu.make_async_copy(ei_hbm.at[0, wid, kk], srcs[b], isems[b]).wait()
        pltpu.make_async_copy(ei_hbm.at[1, wid, kk], dsts[b], isems[b]).wait()

    def g_issue(kk, b):
        pltpu.async_copy(x_hbm.at[srcs[b]], rows_v.at[b], gsems[b])

    def g_wait(kk, b):
        pltpu.make_async_copy(x_hbm.at[srcs[b]], rows_v.at[b],
                              gsems[b]).wait()

    _edge_pipeline(GNCH, GNB, i_issue, i_wait, g_issue, g_wait, rows_v,
                   dsts, g_sh, ssems)

    plsc.subcore_barrier()
    _copy_out(g_sh, g_out, c, s)


@functools.partial(
    pl.kernel,
    out_type=jax.ShapeDtypeStruct((NC, N, D), jnp.float32),
    mesh=_vector_mesh,
    scratch_types=[
        pltpu.VMEM((CHUNK,), jnp.int32),
        pltpu.VMEM((CHUNK,), jnp.int32),
        pltpu.VMEM((CHUNK,), jnp.int32),
        pltpu.VMEM((NBUF, CHUNK, D), jnp.float32),
        pltpu.VMEM_SHARED((N_SH, D), jnp.float32),
        pltpu.SemaphoreType.DMA,
        pltpu.SemaphoreType.DMA,
        pltpu.SemaphoreType.DMA,
        pltpu.SemaphoreType.DMA,
        pltpu.SemaphoreType.DMA,
        pltpu.SemaphoreType.DMA,
        pltpu.SemaphoreType.DMA,
        pltpu.SemaphoreType.DMA,
        pltpu.SemaphoreType.DMA,
    ],
)
def _sc_secnt(ea_hbm, ei_hbm, se_out,
              d0, d1, d2, rows_v, se_sh,
              is0, is1, is2, gs0, gs1, gs2, ss0, ss1, ss2):
    c = lax.axis_index("c")
    s = lax.axis_index("s")
    wid = c * NS + s
    dsts = (d0, d1, d2)
    isems = (is0, is1, is2)
    gsems = (gs0, gs1, gs2)
    ssems = (ss0, ss1, ss2)

    _zero_rows(rows_v.at[0])
    _zero_accum(rows_v.at[0], se_sh, s)
    plsc.subcore_barrier()

    def i_issue(kk, b):
        pltpu.async_copy(ei_hbm.at[1, wid, kk], dsts[b], isems[b])

    def i_wait(kk, b):
        pltpu.make_async_copy(ei_hbm.at[1, wid, kk], dsts[b], isems[b]).wait()

    def l_issue(kk, b):
        base = wid * EW + kk * CHUNK
        pltpu.async_copy(ea_hbm.at[pl.ds(base, CHUNK)], rows_v.at[b],
                         gsems[b])

    def l_wait(kk, b):
        base = wid * EW + kk * CHUNK
        pltpu.make_async_copy(ea_hbm.at[pl.ds(base, CHUNK)], rows_v.at[b],
                              gsems[b]).wait()

    _edge_pipeline(NCHUNK, NBUF, i_issue, i_wait, l_issue, l_wait, rows_v,
                   dsts, se_sh, ssems)

    plsc.subcore_barrier()
    _copy_out(se_sh, se_out, c, s)


ROWS_BLK = 1000


def _tc_layer_body(x_ref, gp_ref, sep_ref, wmx_ref, wme_ref, bm_ref,
                   wax_ref, waa_ref, ba_ref, o_ref):
    g = gp_ref[0] + gp_ref[1]
    sec = sep_ref[0] + sep_ref[1]
    se = sec[:, :DE]
    cnt = sec[:, DE:DE + 1]
    msum = jnp.dot(g, wmx_ref[...], preferred_element_type=jnp.float32)
    msum += jnp.dot(se, wme_ref[...], preferred_element_type=jnp.float32)
    msum += cnt * bm_ref[...]
    aggr = msum / jnp.maximum(cnt, 1.0)
    h = jnp.dot(x_ref[...], wax_ref[...], preferred_element_type=jnp.float32)
    h += jnp.dot(aggr, waa_ref[...], preferred_element_type=jnp.float32)
    h += ba_ref[...]
    o_ref[...] = jnp.maximum(h, 0.0)


_tc_layer = pl.pallas_call(
    _tc_layer_body,
    grid=(N // ROWS_BLK,),
    in_specs=[
        pl.BlockSpec((ROWS_BLK, D), lambda i: (i, 0)),
        pl.BlockSpec((NC, ROWS_BLK, D), lambda i: (0, i, 0)),
        pl.BlockSpec((NC, ROWS_BLK, D), lambda i: (0, i, 0)),
        pl.BlockSpec((D, D), lambda i: (0, 0)),
        pl.BlockSpec((DE, D), lambda i: (0, 0)),
        pl.BlockSpec((1, D), lambda i: (0, 0)),
        pl.BlockSpec((D, D), lambda i: (0, 0)),
        pl.BlockSpec((D, D), lambda i: (0, 0)),
        pl.BlockSpec((1, D), lambda i: (0, 0)),
    ],
    out_specs=pl.BlockSpec((ROWS_BLK, D), lambda i: (i, 0)),
    out_shape=jax.ShapeDtypeStruct((N, D), jnp.float32),
)


def kernel(x, edge_index, edge_attr, W_msg1, b_msg1, W_apply1, b_apply1,
           W_msg2, b_msg2, W_apply2, b_apply2):
    ei_g = edge_index.reshape(2, NW, GNCH, GCH)
    ei_s = edge_index.reshape(2, NW, NCHUNK, CHUNK)
    ea_aug = jnp.concatenate(
        [edge_attr, jnp.ones((E, 1), jnp.float32),
         jnp.zeros((E, D - DE - 1), jnp.float32)], axis=1)

    g1p = _sc_gather_segsum(x, ei_g)
    sep = _sc_secnt(ea_aug, ei_s)
    h1 = _tc_layer(x, g1p, sep,
                   W_msg1[:D], W_msg1[D:], b_msg1.reshape(1, D),
                   W_apply1[:D], W_apply1[D:], b_apply1.reshape(1, D))
    g2p = _sc_gather_segsum(h1, ei_g)
    out = _tc_layer(h1, g2p, sep,
                    W_msg2[:D], W_msg2[D:], b_msg2.reshape(1, D),
                    W_apply2[:D], W_apply2[D:], b_apply2.reshape(1, D))
    return out


# speedup vs baseline: 1.0711x; 1.0711x over previous
import functools

import jax
import jax.numpy as jnp
from jax import lax
from jax.experimental import pallas as pl
from jax.experimental.pallas import tpu as pltpu
from jax.experimental.pallas import tpu_sc as plsc

N = 10000
E = 320000
D = 128
DE = 16

NC = 2
NS = 16
NW = NC * NS
EW = E // NW
GCH = 50
GNB = 5
GNCH = EW // GCH
CHUNK = 80
NCHUNK = EW // CHUNK
NBUF = 3
N_SH = 10080
ZCH = 80
NZ = N_SH // ZCH
OUT_CHUNK = 80
NOUT = N // OUT_CHUNK

_vector_mesh = plsc.VectorSubcoreMesh(core_axis_name="c", subcore_axis_name="s")


def _zero_rows(ref):
    zv = jnp.zeros((16,), jnp.float32)

    @pl.loop(0, ref.shape[0])
    def _(i):
        @pl.loop(0, D, step=16)
        def _(j):
            ref[i, pl.ds(j, 16)] = zv


def _zero_accum(zb, sh_ref, s):
    @pl.loop(0, pl.cdiv(NZ, NS))
    def _(k):
        ci = k * NS + s

        @pl.when(ci < NZ)
        def _():
            pltpu.sync_copy(zb, sh_ref.at[pl.ds(ci * ZCH, ZCH)])


def _copy_out(sh_ref, out_ref, c, s):
    @pl.loop(0, pl.cdiv(NOUT, NS))
    def _(k):
        ci = k * NS + s

        @pl.when(ci < NOUT)
        def _():
            row = ci * OUT_CHUNK
            pltpu.sync_copy(sh_ref.at[pl.ds(row, OUT_CHUNK)],
                            out_ref.at[c, pl.ds(row, OUT_CHUNK)])


def _edge_pipeline(nchunk, nbuf, idx_issue, idx_wait, load_issue, load_wait,
                   rows_v, dsts, g_sh, ssems):
    def s_issue(kk, b):
        pltpu.async_copy(rows_v.at[b], g_sh.at[dsts[b]], ssems[b], add=True)

    def s_wait(kk, b):
        pltpu.make_async_copy(rows_v.at[b], g_sh.at[dsts[b]], ssems[b]).wait()

    for b in range(nbuf):
        idx_issue(b, b)

    @pl.loop(0, pl.cdiv(nchunk, nbuf))
    def _(g):
        for b in range(nbuf):
            kk = g * nbuf + b

            @pl.when(kk < nchunk)
            def _():
                idx_wait(kk, b)
                load_issue(kk, b)

        for b in range(nbuf):
            kk = g * nbuf + b

            @pl.when(kk < nchunk)
            def _():
                load_wait(kk, b)
                s_issue(kk, b)

        for b in range(nbuf):
            kk = g * nbuf + b
            nxt = kk + nbuf

            @pl.when(kk < nchunk)
            def _():
                s_wait(kk, b)

                @pl.when(nxt < nchunk)
                def _():
                    idx_issue(nxt, b)


@functools.partial(
    pl.kernel,
    out_type=jax.ShapeDtypeStruct((NC, N, D), jnp.float32),
    mesh=_vector_mesh,
    scratch_types=[
        pltpu.VMEM((GCH,), jnp.int32),
        pltpu.VMEM((GCH,), jnp.int32),
        pltpu.VMEM((GCH,), jnp.int32),
        pltpu.VMEM((GCH,), jnp.int32),
        pltpu.VMEM((GCH,), jnp.int32),
        pltpu.VMEM((GCH,), jnp.int32),
        pltpu.VMEM((GCH,), jnp.int32),
        pltpu.VMEM((GCH,), jnp.int32),
        pltpu.VMEM((GCH,), jnp.int32),
        pltpu.VMEM((GCH,), jnp.int32),
        pltpu.VMEM((GNB, GCH, D), jnp.float32),
        pltpu.VMEM_SHARED((N_SH, D), jnp.float32),
        pltpu.SemaphoreType.DMA,
        pltpu.SemaphoreType.DMA,
        pltpu.SemaphoreType.DMA,
        pltpu.SemaphoreType.DMA,
        pltpu.SemaphoreType.DMA,
        pltpu.SemaphoreType.DMA,
        pltpu.SemaphoreType.DMA,
        pltpu.SemaphoreType.DMA,
        pltpu.SemaphoreType.DMA,
        pltpu.SemaphoreType.DMA,
        pltpu.SemaphoreType.DMA,
        pltpu.SemaphoreType.DMA,
        pltpu.SemaphoreType.DMA,
        pltpu.SemaphoreType.DMA,
        pltpu.SemaphoreType.DMA,
    ],
)
def _sc_gather_segsum(x_hbm, ei_hbm, g_out,
                      s0, s1, s2, s3, s4, d0, d1, d2, d3, d4, rows_v, g_sh,
                      is0, is1, is2, is3, is4, gs0, gs1, gs2, gs3, gs4,
                      ss0, ss1, ss2, ss3, ss4):
    c = lax.axis_index("c")
    s = lax.axis_index("s")
    wid = c * NS + s
    srcs = (s0, s1, s2, s3, s4)
    dsts = (d0, d1, d2, d3, d4)
    isems = (is0, is1, is2, is3, is4)
    gsems = (gs0, gs1, gs2, gs3, gs4)
    ssems = (ss0, ss1, ss2, ss3, ss4)

    _zero_rows(rows_v.at[0])
    _zero_accum(rows_v.at[0].at[pl.ds(0, ZCH)], g_sh, s)
    plsc.subcore_barrier()

    def i_issue(kk, b):
        pltpu.async_copy(ei_hbm.at[0, wid, kk], srcs[b], isems[b])
        pltpu.async_copy(ei_hbm.at[1, wid, kk], dsts[b], isems[b])

    def i_wait(kk, b):
        pltpu.make_async_copy(ei_hbm.at[0, wid, kk], srcs[b], isems[b]).wait()
        pltpu.make_async_copy(ei_hbm.at[1, wid, kk], dsts[b], isems[b]).wait()

    def g_issue(kk, b):
        pltpu.async_copy(x_hbm.at[srcs[b]], rows_v.at[b], gsems[b])

    def g_wait(kk, b):
        pltpu.make_async_copy(x_hbm.at[srcs[b]], rows_v.at[b],
                              gsems[b]).wait()

    _edge_pipeline(GNCH, GNB, i_issue, i_wait, g_issue, g_wait, rows_v,
                   dsts, g_sh, ssems)

    plsc.subcore_barrier()
    _copy_out(g_sh, g_out, c, s)


@functools.partial(
    pl.kernel,
    out_type=jax.ShapeDtypeStruct((NC, N, D), jnp.float32),
    mesh=_vector_mesh,
    scratch_types=[
        pltpu.VMEM((CHUNK,), jnp.int32),
        pltpu.VMEM((CHUNK,), jnp.int32),
        pltpu.VMEM((CHUNK,), jnp.int32),
        pltpu.VMEM((NBUF, CHUNK, D), jnp.float32),
        pltpu.VMEM_SHARED((N_SH, D), jnp.float32),
        pltpu.SemaphoreType.DMA,
        pltpu.SemaphoreType.DMA,
        pltpu.SemaphoreType.DMA,
        pltpu.SemaphoreType.DMA,
        pltpu.SemaphoreType.DMA,
        pltpu.SemaphoreType.DMA,
        pltpu.SemaphoreType.DMA,
        pltpu.SemaphoreType.DMA,
        pltpu.SemaphoreType.DMA,
    ],
)
def _sc_secnt(ea_hbm, ei_hbm, se_out,
              d0, d1, d2, rows_v, se_sh,
              is0, is1, is2, gs0, gs1, gs2, ss0, ss1, ss2):
    c = lax.axis_index("c")
    s = lax.axis_index("s")
    wid = c * NS + s
    dsts = (d0, d1, d2)
    isems = (is0, is1, is2)
    gsems = (gs0, gs1, gs2)
    ssems = (ss0, ss1, ss2)

    _zero_rows(rows_v.at[0])
    _zero_accum(rows_v.at[0], se_sh, s)
    plsc.subcore_barrier()

    def i_issue(kk, b):
        pltpu.async_copy(ei_hbm.at[1, wid, kk], dsts[b], isems[b])

    def i_wait(kk, b):
        pltpu.make_async_copy(ei_hbm.at[1, wid, kk], dsts[b], isems[b]).wait()

    def l_issue(kk, b):
        base = wid * EW + kk * CHUNK
        pltpu.async_copy(ea_hbm.at[pl.ds(base, CHUNK)], rows_v.at[b],
                         gsems[b])

    def l_wait(kk, b):
        base = wid * EW + kk * CHUNK
        pltpu.make_async_copy(ea_hbm.at[pl.ds(base, CHUNK)], rows_v.at[b],
                              gsems[b]).wait()

    _edge_pipeline(NCHUNK, NBUF, i_issue, i_wait, l_issue, l_wait, rows_v,
                   dsts, se_sh, ssems)

    plsc.subcore_barrier()
    _copy_out(se_sh, se_out, c, s)


ROWS_BLK = 1000


def _tc_layer_body(x_ref, gp_ref, sep_ref, wmx_ref, wme_ref, bm_ref,
                   wax_ref, waa_ref, ba_ref, o_ref):
    g = gp_ref[0] + gp_ref[1]
    sec = sep_ref[0] + sep_ref[1]
    se = sec[:, :DE]
    cnt = sec[:, DE:DE + 1]
    msum = jnp.dot(g, wmx_ref[...], preferred_element_type=jnp.float32)
    msum += jnp.dot(se, wme_ref[...], preferred_element_type=jnp.float32)
    msum += cnt * bm_ref[...]
    aggr = msum / jnp.maximum(cnt, 1.0)
    h = jnp.dot(x_ref[...], wax_ref[...], preferred_element_type=jnp.float32)
    h += jnp.dot(aggr, waa_ref[...], preferred_element_type=jnp.float32)
    h += ba_ref[...]
    o_ref[...] = jnp.maximum(h, 0.0)


_tc_layer = pl.pallas_call(
    _tc_layer_body,
    grid=(N // ROWS_BLK,),
    in_specs=[
        pl.BlockSpec((ROWS_BLK, D), lambda i: (i, 0)),
        pl.BlockSpec((NC, ROWS_BLK, D), lambda i: (0, i, 0)),
        pl.BlockSpec((NC, ROWS_BLK, D), lambda i: (0, i, 0)),
        pl.BlockSpec((D, D), lambda i: (0, 0)),
        pl.BlockSpec((DE, D), lambda i: (0, 0)),
        pl.BlockSpec((1, D), lambda i: (0, 0)),
        pl.BlockSpec((D, D), lambda i: (0, 0)),
        pl.BlockSpec((D, D), lambda i: (0, 0)),
        pl.BlockSpec((1, D), lambda i: (0, 0)),
    ],
    out_specs=pl.BlockSpec((ROWS_BLK, D), lambda i: (i, 0)),
    out_shape=jax.ShapeDtypeStruct((N, D), jnp.float32),
)


def kernel(x, edge_index, edge_attr, W_msg1, b_msg1, W_apply1, b_apply1,
           W_msg2, b_msg2, W_apply2, b_apply2):
    ei_g = edge_index.reshape(2, NW, GNCH, GCH)
    ei_s = edge_index.reshape(2, NW, NCHUNK, CHUNK)
    ea_aug = jnp.concatenate(
        [edge_attr, jnp.ones((E, 1), jnp.float32),
         jnp.zeros((E, D - DE - 1), jnp.float32)], axis=1)

    g1p = _sc_gather_segsum(x, ei_g)
    sep = _sc_secnt(ea_aug, ei_s)
    h1 = _tc_layer(x, g1p, sep,
                   W_msg1[:D], W_msg1[D:], b_msg1.reshape(1, D),
                   W_apply1[:D], W_apply1[D:], b_apply1.reshape(1, D))
    g2p = _sc_gather_segsum(h1, ei_g)
    out = _tc_layer(h1, g2p, sep,
                    W_msg2[:D], W_msg2[D:], b_msg2.reshape(1, D),
                    W_apply2[:D], W_apply2[D:], b_apply2.reshape(1, D))
    return out
